# initial kernel scaffold (unmeasured)
import jax
import jax.numpy as jnp
from jax import lax
from jax.experimental import pallas as pl
from jax.experimental.pallas import tpu as pltpu

N_DEV = 4
SQ = 1024
SKV_LOCAL = 1024
SKV = N_DEV * SKV_LOCAL
HQ_LOCAL = 8
DH = 128
DMODEL = 1024
SCALE = 0.08838834764831843


def kernel(x, Wq, K_ext, V_ext, Wo):
    xb = x[0].astype(jnp.bfloat16)
    wqb = Wq.astype(jnp.bfloat16)
    kb = K_ext[0].astype(jnp.bfloat16)
    vb = V_ext[0].astype(jnp.bfloat16)
    wob = Wo.astype(jnp.bfloat16)

    def body(x_ref, wq_ref, k_ref, v_ref, wo_ref, out_ref,
             kstage, vstage, k_all, v_all, p_all,
             k_send, v_send, p_send, k_recv, v_recv, p_recv):
        my = lax.axis_index("i")

        barrier = pltpu.get_barrier_semaphore()
        for o in range(1, N_DEV):
            peer = lax.rem(my + o, N_DEV)
            pl.semaphore_signal(barrier, inc=1, device_id=(peer,),
                                device_id_type=pl.DeviceIdType.MESH)
        pl.semaphore_wait(barrier, N_DEV - 1)

        for g in range(N_DEV):
            kstage[g] = k_ref[:, g * HQ_LOCAL:(g + 1) * HQ_LOCAL, :]
            vstage[g] = v_ref[:, g * HQ_LOCAL:(g + 1) * HQ_LOCAL, :]
        k_all[pl.ds(my, 1)] = kstage[pl.ds(my, 1)]
        v_all[pl.ds(my, 1)] = vstage[pl.ds(my, 1)]

        kv_pairs = ((kstage, k_all, k_send, k_recv),
                    (vstage, v_all, v_send, v_recv))

        sends = []
        for o in range(1, N_DEV):
            t = lax.rem(my + o, N_DEV)
            for (stage, all_, ssem, rsem) in kv_pairs:
                rdma = pltpu.make_async_remote_copy(
                    src_ref=stage.at[pl.ds(t, 1)],
                    dst_ref=all_.at[pl.ds(my, 1)],
                    send_sem=ssem.at[o],
                    recv_sem=rsem.at[o],
                    device_id=(t,),
                    device_id_type=pl.DeviceIdType.MESH,
                )
                rdma.start()
                sends.append(rdma)

        q = lax.dot(x_ref[...], wq_ref[...],
                    preferred_element_type=jnp.float32)
        q = (q * SCALE).astype(jnp.bfloat16)

        for o in range(1, N_DEV):
            s = lax.rem(my + N_DEV - o, N_DEV)
            for (stage, all_, ssem, rsem) in kv_pairs:
                recv = pltpu.make_async_remote_copy(
                    src_ref=stage.at[pl.ds(s, 1)],
                    dst_ref=all_.at[pl.ds(s, 1)],
                    send_sem=ssem.at[o],
                    recv_sem=rsem.at[o],
                    device_id=(s,),
                    device_id_type=pl.DeviceIdType.MESH,
                )
                recv.wait_recv()

        qb4 = (lax.broadcasted_iota(jnp.int32, (SQ, SKV), 0) // 64) % 4
        kb4 = (lax.broadcasted_iota(jnp.int32, (SQ, SKV), 1) // 64) % 4
        mask = qb4 == kb4

        acc = jnp.zeros((SQ, DMODEL), jnp.float32)
        for h in range(HQ_LOCAL):
            q_h = q[:, h * DH:(h + 1) * DH]
            k_h = k_all[:, :, h, :].reshape(SKV, DH)
            s_ = lax.dot_general(q_h, k_h, (((1,), (1,)), ((), ())),
                                 preferred_element_type=jnp.float32)
            s_ = jnp.where(mask, s_, -1e9)
            m = jnp.max(s_, axis=1, keepdims=True)
            w = jnp.exp(s_ - m)
            denom = jnp.sum(w, axis=1, keepdims=True)
            wn = (w / denom).astype(jnp.bfloat16)
            v_h = v_all[:, :, h, :].reshape(SKV, DH)
            ctx = lax.dot_general(wn, v_h, (((1,), (0,)), ((), ())),
                                  preferred_element_type=jnp.float32)
            ctx = ctx.astype(jnp.bfloat16)
            wo_h = wo_ref[h * DH:(h + 1) * DH, :]
            acc = acc + lax.dot(ctx, wo_h,
                                preferred_element_type=jnp.float32)

        p_all[pl.ds(my, 1)] = acc.astype(jnp.bfloat16)[None]
        for o in range(1, N_DEV):
            t = lax.rem(my + o, N_DEV)
            rdma = pltpu.make_async_remote_copy(
                src_ref=p_all.at[pl.ds(my, 1)],
                dst_ref=p_all.at[pl.ds(my, 1)],
                send_sem=p_send.at[o],
                recv_sem=p_recv.at[o],
                device_id=(t,),
                device_id_type=pl.DeviceIdType.MESH,
            )
            rdma.start()
            sends.append(rdma)

        total = acc
        for o in range(1, N_DEV):
            s = lax.rem(my + N_DEV - o, N_DEV)
            recv = pltpu.make_async_remote_copy(
                src_ref=p_all.at[pl.ds(s, 1)],
                dst_ref=p_all.at[pl.ds(s, 1)],
                send_sem=p_send.at[o],
                recv_sem=p_recv.at[o],
                device_id=(s,),
                device_id_type=pl.DeviceIdType.MESH,
            )
            recv.wait_recv()
            total = total + p_all[pl.ds(s, 1)].reshape(SQ, DMODEL).astype(
                jnp.float32)

        out_ref[...] = total

        for r in sends:
            r.wait_send()

    out = pl.pallas_call(
        body,
        out_shape=jax.ShapeDtypeStruct((SQ, DMODEL), jnp.float32),
        in_specs=[pl.BlockSpec(memory_space=pltpu.VMEM)] * 5,
        out_specs=pl.BlockSpec(memory_space=pltpu.VMEM),
        scratch_shapes=[
            pltpu.VMEM((N_DEV, SKV_LOCAL, HQ_LOCAL, DH), jnp.bfloat16),
            pltpu.VMEM((N_DEV, SKV_LOCAL, HQ_LOCAL, DH), jnp.bfloat16),
            pltpu.VMEM((N_DEV, SKV_LOCAL, HQ_LOCAL, DH), jnp.bfloat16),
            pltpu.VMEM((N_DEV, SKV_LOCAL, HQ_LOCAL, DH), jnp.bfloat16),
            pltpu.VMEM((N_DEV, SQ, DMODEL), jnp.bfloat16),
            pltpu.SemaphoreType.DMA((N_DEV,)),
            pltpu.SemaphoreType.DMA((N_DEV,)),
            pltpu.SemaphoreType.DMA((N_DEV,)),
            pltpu.SemaphoreType.DMA((N_DEV,)),
            pltpu.SemaphoreType.DMA((N_DEV,)),
            pltpu.SemaphoreType.DMA((N_DEV,)),
        ],
        compiler_params=pltpu.CompilerParams(collective_id=0),
    )(xb, wqb, kb, vb, wob)
    return out[None]


# baseline (device time: 228680 ns/iter reference)
import jax
import jax.numpy as jnp
from jax import lax
from jax.experimental import pallas as pl
from jax.experimental.pallas import tpu as pltpu

N_DEV = 4
SQ = 1024
SKV_LOCAL = 1024
SKV = N_DEV * SKV_LOCAL
HQ_LOCAL = 8
DH = 128
DMODEL = 1024
SCALE = 0.08838834764831843


def kernel(x, Wq, K_ext, V_ext, Wo):
    xb = x[0].astype(jnp.bfloat16)
    wqb = Wq.astype(jnp.bfloat16)
    kb = K_ext[0].astype(jnp.bfloat16)
    vb = V_ext[0].astype(jnp.bfloat16)
    wob = Wo.astype(jnp.bfloat16)

    def body(x_ref, wq_ref, k_ref, v_ref, wo_ref, out_ref,
             k_all, v_all, p_all, q_ref, ctx_ref,
             k_send, v_send, p_send, k_recv, v_recv, p_recv, local_sem):
        my = lax.axis_index("i")

        k_local = pltpu.make_async_copy(
            k_ref.at[:, pl.ds(my * HQ_LOCAL, HQ_LOCAL), :],
            k_all.at[my], local_sem.at[0])
        k_local.start()
        v_local = pltpu.make_async_copy(
            v_ref.at[:, pl.ds(my * HQ_LOCAL, HQ_LOCAL), :],
            v_all.at[my], local_sem.at[1])
        v_local.start()

        barrier = pltpu.get_barrier_semaphore()
        for o in range(1, N_DEV):
            peer = lax.rem(my + o, N_DEV)
            pl.semaphore_signal(barrier, inc=1, device_id=(peer,),
                                device_id_type=pl.DeviceIdType.MESH)
        pl.semaphore_wait(barrier, N_DEV - 1)

        sends = []
        for o in range(1, N_DEV):
            t = lax.rem(my + o, N_DEV)
            for (src, all_, ssem, rsem) in ((k_ref, k_all, k_send, k_recv),
                                            (v_ref, v_all, v_send, v_recv)):
                rdma = pltpu.make_async_remote_copy(
                    src_ref=src.at[:, pl.ds(t * HQ_LOCAL, HQ_LOCAL), :],
                    dst_ref=all_.at[my],
                    send_sem=ssem.at[o],
                    recv_sem=rsem.at[o],
                    device_id=(t,),
                    device_id_type=pl.DeviceIdType.MESH,
                )
                rdma.start()
                sends.append(rdma)

        q_ref[...] = (lax.dot(x_ref[...], wq_ref[...],
                              preferred_element_type=jnp.float32)
                      * SCALE).astype(jnp.bfloat16)

        k_local.wait()
        v_local.wait()
        for o in range(1, N_DEV):
            s = lax.rem(my + N_DEV - o, N_DEV)
            for (src, all_, ssem, rsem) in ((k_ref, k_all, k_send, k_recv),
                                            (v_ref, v_all, v_send, v_recv)):
                recv = pltpu.make_async_remote_copy(
                    src_ref=src.at[:, pl.ds(s * HQ_LOCAL, HQ_LOCAL), :],
                    dst_ref=all_.at[s],
                    send_sem=ssem.at[o],
                    recv_sem=rsem.at[o],
                    device_id=(s,),
                    device_id_type=pl.DeviceIdType.MESH,
                )
                recv.wait_recv()

        for h in range(HQ_LOCAL):
            qh = q_ref[:, h * DH:(h + 1) * DH].reshape(4, 4, 64, DH)
            ka = k_all[:, :, h, :].reshape(N_DEV, 4, 4, 64, DH)
            va = v_all[:, :, h, :].reshape(N_DEV, 4, 4, 64, DH)
            for g in range(4):
                qg = qh[:, g].reshape(256, DH)
                kg = ka[:, :, g].reshape(SKV // 4, DH)
                vg = va[:, :, g].reshape(SKV // 4, DH)
                s_ = lax.dot_general(qg, kg, (((1,), (1,)), ((), ())),
                                     preferred_element_type=jnp.float32)
                mx = jnp.max(s_, axis=1, keepdims=True)
                w = jnp.exp(s_ - mx)
                wn = (w / jnp.sum(w, axis=1, keepdims=True)).astype(
                    jnp.bfloat16)
                ctxg = lax.dot_general(wn, vg, (((1,), (0,)), ((), ())),
                                       preferred_element_type=jnp.float32)
                ctxg = ctxg.astype(jnp.bfloat16).reshape(4, 64, DH)
                for c in range(4):
                    ctx_ref[pl.ds(256 * c + 64 * g, 64),
                            pl.ds(h * DH, DH)] = ctxg[c]

        p_all[pl.ds(my, 1)] = lax.dot(
            ctx_ref[...], wo_ref[...],
            preferred_element_type=jnp.float32).astype(jnp.bfloat16)[None]
        for o in range(1, N_DEV):
            t = lax.rem(my + o, N_DEV)
            rdma = pltpu.make_async_remote_copy(
                src_ref=p_all.at[my],
                dst_ref=p_all.at[my],
                send_sem=p_send.at[o],
                recv_sem=p_recv.at[o],
                device_id=(t,),
                device_id_type=pl.DeviceIdType.MESH,
            )
            rdma.start()
            sends.append(rdma)

        out_ref[...] = p_all[pl.ds(my, 1)].reshape(SQ, DMODEL).astype(
            jnp.float32)
        for o in range(1, N_DEV):
            s = lax.rem(my + N_DEV - o, N_DEV)
            recv = pltpu.make_async_remote_copy(
                src_ref=p_all.at[s],
                dst_ref=p_all.at[s],
                send_sem=p_send.at[o],
                recv_sem=p_recv.at[o],
                device_id=(s,),
                device_id_type=pl.DeviceIdType.MESH,
            )
            recv.wait_recv()
            out_ref[...] = out_ref[...] + p_all[pl.ds(s, 1)].reshape(
                SQ, DMODEL).astype(jnp.float32)

        for r in sends:
            r.wait_send()

    out = pl.pallas_call(
        body,
        out_shape=jax.ShapeDtypeStruct((SQ, DMODEL), jnp.float32),
        in_specs=[
            pl.BlockSpec(memory_space=pltpu.VMEM),
            pl.BlockSpec(memory_space=pltpu.VMEM),
            pl.BlockSpec(memory_space=pltpu.MemorySpace.HBM),
            pl.BlockSpec(memory_space=pltpu.MemorySpace.HBM),
            pl.BlockSpec(memory_space=pltpu.VMEM),
        ],
        out_specs=pl.BlockSpec(memory_space=pltpu.VMEM),
        scratch_shapes=[
            pltpu.VMEM((N_DEV, SKV_LOCAL, HQ_LOCAL, DH), jnp.bfloat16),
            pltpu.VMEM((N_DEV, SKV_LOCAL, HQ_LOCAL, DH), jnp.bfloat16),
            pltpu.VMEM((N_DEV, SQ, DMODEL), jnp.bfloat16),
            pltpu.VMEM((SQ, DMODEL), jnp.bfloat16),
            pltpu.VMEM((SQ, DMODEL), jnp.bfloat16),
            pltpu.SemaphoreType.DMA((N_DEV,)),
            pltpu.SemaphoreType.DMA((N_DEV,)),
            pltpu.SemaphoreType.DMA((N_DEV,)),
            pltpu.SemaphoreType.DMA((N_DEV,)),
            pltpu.SemaphoreType.DMA((N_DEV,)),
            pltpu.SemaphoreType.DMA((N_DEV,)),
            pltpu.SemaphoreType.DMA((2,)),
        ],
        compiler_params=pltpu.CompilerParams(
            collective_id=0, vmem_limit_bytes=54 * 1024 * 1024),
    )(xb, wqb, kb, vb, wob)
    return out[None]


# device time: 193667 ns/iter; 1.1808x vs baseline; 1.1808x over previous
import jax
import jax.numpy as jnp
from jax import lax
from jax.experimental import pallas as pl
from jax.experimental.pallas import tpu as pltpu

N_DEV = 4
SQ = 1024
SKV_LOCAL = 1024
SKV = N_DEV * SKV_LOCAL
HQ_LOCAL = 8
DH = 128
DMODEL = 1024
SCALE = 0.08838834764831843


def kernel(x, Wq, K_ext, V_ext, Wo):
    xb = x[0].astype(jnp.bfloat16)
    wqb = Wq.astype(jnp.bfloat16)
    kb = K_ext[0].astype(jnp.bfloat16)
    vb = V_ext[0].astype(jnp.bfloat16)
    wob = Wo.astype(jnp.bfloat16)

    def body(x_ref, wq_ref, k_ref, v_ref, wo_ref, out_ref,
             k_all, v_all, p_all, q_ref, ctx_ref,
             k_send, v_send, p_send, k_recv, v_recv, p_recv, local_sem):
        my = lax.axis_index("i")

        k_local = pltpu.make_async_copy(
            k_ref.at[:, pl.ds(my * HQ_LOCAL, HQ_LOCAL), :],
            k_all.at[my], local_sem.at[0])
        k_local.start()
        v_local = pltpu.make_async_copy(
            v_ref.at[:, pl.ds(my * HQ_LOCAL, HQ_LOCAL), :],
            v_all.at[my], local_sem.at[1])
        v_local.start()

        barrier = pltpu.get_barrier_semaphore()
        for o in range(1, N_DEV):
            peer = lax.rem(my + o, N_DEV)
            pl.semaphore_signal(barrier, inc=1, device_id=(peer,),
                                device_id_type=pl.DeviceIdType.MESH)
        pl.semaphore_wait(barrier, N_DEV - 1)

        sends = []
        for o in range(1, N_DEV):
            t = lax.rem(my + o, N_DEV)
            for (src, all_, ssem, rsem) in ((k_ref, k_all, k_send, k_recv),
                                            (v_ref, v_all, v_send, v_recv)):
                rdma = pltpu.make_async_remote_copy(
                    src_ref=src.at[:, pl.ds(t * HQ_LOCAL, HQ_LOCAL), :],
                    dst_ref=all_.at[my],
                    send_sem=ssem.at[o],
                    recv_sem=rsem.at[o],
                    device_id=(t,),
                    device_id_type=pl.DeviceIdType.MESH,
                )
                rdma.start()
                sends.append(rdma)

        q_ref[...] = (lax.dot(x_ref[...], wq_ref[...],
                              preferred_element_type=jnp.float32)
                      * SCALE).astype(jnp.bfloat16)

        k_local.wait()
        v_local.wait()
        for o in (1, 3, 2):
            s = lax.rem(my + N_DEV - o, N_DEV)
            for (src, all_, ssem, rsem) in ((k_ref, k_all, k_send, k_recv),
                                            (v_ref, v_all, v_send, v_recv)):
                recv = pltpu.make_async_remote_copy(
                    src_ref=src.at[:, pl.ds(s * HQ_LOCAL, HQ_LOCAL), :],
                    dst_ref=all_.at[s],
                    send_sem=ssem.at[o],
                    recv_sem=rsem.at[o],
                    device_id=(s,),
                    device_id_type=pl.DeviceIdType.MESH,
                )
                recv.wait_recv()

        for g in range(4):
            for h in range(HQ_LOCAL):
                qh = q_ref[:, h * DH:(h + 1) * DH].reshape(4, 4, 64, DH)
                ka = k_all[:, :, h, :].reshape(N_DEV, 4, 4, 64, DH)
                va = v_all[:, :, h, :].reshape(N_DEV, 4, 4, 64, DH)
                qg = qh[:, g].reshape(256, DH)
                kg = ka[:, :, g].reshape(SKV // 4, DH)
                vg = va[:, :, g].reshape(SKV // 4, DH)
                s_ = lax.dot_general(qg, kg, (((1,), (1,)), ((), ())),
                                     preferred_element_type=jnp.float32)
                mx = jnp.max(s_, axis=1, keepdims=True)
                w = jnp.exp(s_ - mx)
                wn = (w / jnp.sum(w, axis=1, keepdims=True)).astype(
                    jnp.bfloat16)
                ctxg = lax.dot_general(wn, vg, (((1,), (0,)), ((), ())),
                                       preferred_element_type=jnp.float32)
                ctx_ref[pl.ds(256 * g, 256), pl.ds(h * DH, DH)] = (
                    ctxg.astype(jnp.bfloat16))
            p_all[pl.ds(my, 1), pl.ds(256 * g, 256), :] = lax.dot(
                ctx_ref[pl.ds(256 * g, 256), :], wo_ref[...],
                preferred_element_type=jnp.float32).astype(jnp.bfloat16)[None]
            for o in range(1, N_DEV):
                t = lax.rem(my + o, N_DEV)
                rdma = pltpu.make_async_remote_copy(
                    src_ref=p_all.at[my, pl.ds(256 * g, 256)],
                    dst_ref=p_all.at[my, pl.ds(256 * g, 256)],
                    send_sem=p_send.at[o, g],
                    recv_sem=p_recv.at[o, g],
                    device_id=(t,),
                    device_id_type=pl.DeviceIdType.MESH,
                )
                rdma.start()
                sends.append(rdma)

        for o in (1, 3, 2):
            s = lax.rem(my + N_DEV - o, N_DEV)
            for g in range(4):
                recv = pltpu.make_async_remote_copy(
                    src_ref=p_all.at[s, pl.ds(256 * g, 256)],
                    dst_ref=p_all.at[s, pl.ds(256 * g, 256)],
                    send_sem=p_send.at[o, g],
                    recv_sem=p_recv.at[o, g],
                    device_id=(s,),
                    device_id_type=pl.DeviceIdType.MESH,
                )
                recv.wait_recv()

        for c in range(4):
            for g in range(4):
                src_rows = pl.ds(256 * g + 64 * c, 64)
                acc = (p_all[0, src_rows, :].astype(jnp.float32)
                       + p_all[1, src_rows, :].astype(jnp.float32)
                       + p_all[2, src_rows, :].astype(jnp.float32)
                       + p_all[3, src_rows, :].astype(jnp.float32))
                out_ref[pl.ds(256 * c + 64 * g, 64), :] = acc

        for r in sends:
            r.wait_send()

    out = pl.pallas_call(
        body,
        out_shape=jax.ShapeDtypeStruct((SQ, DMODEL), jnp.float32),
        in_specs=[
            pl.BlockSpec(memory_space=pltpu.VMEM),
            pl.BlockSpec(memory_space=pltpu.VMEM),
            pl.BlockSpec(memory_space=pltpu.MemorySpace.HBM),
            pl.BlockSpec(memory_space=pltpu.MemorySpace.HBM),
            pl.BlockSpec(memory_space=pltpu.VMEM),
        ],
        out_specs=pl.BlockSpec(memory_space=pltpu.VMEM),
        scratch_shapes=[
            pltpu.VMEM((N_DEV, SKV_LOCAL, HQ_LOCAL, DH), jnp.bfloat16),
            pltpu.VMEM((N_DEV, SKV_LOCAL, HQ_LOCAL, DH), jnp.bfloat16),
            pltpu.VMEM((N_DEV, SQ, DMODEL), jnp.bfloat16),
            pltpu.VMEM((SQ, DMODEL), jnp.bfloat16),
            pltpu.VMEM((SQ, DMODEL), jnp.bfloat16),
            pltpu.SemaphoreType.DMA((N_DEV,)),
            pltpu.SemaphoreType.DMA((N_DEV,)),
            pltpu.SemaphoreType.DMA((N_DEV, 4)),
            pltpu.SemaphoreType.DMA((N_DEV,)),
            pltpu.SemaphoreType.DMA((N_DEV,)),
            pltpu.SemaphoreType.DMA((N_DEV, 4)),
            pltpu.SemaphoreType.DMA((2,)),
        ],
        compiler_params=pltpu.CompilerParams(
            collective_id=0, vmem_limit_bytes=54 * 1024 * 1024),
    )(xb, wqb, kb, vb, wob)
    return out[None]
